# Initial kernel scaffold; baseline (speedup 1.0000x reference)
import jax
import jax.numpy as jnp
from jax.experimental import pallas as pl


def kernel(node_seq, static_feat, adj_vals, Wf, bf, gf, bfn, Wg0, bg0, Wc0, bc0, gn0, bn0, Wg1, bg1, Wc1, bc1, gn1, bn1, edge_index):
    raise NotImplementedError("write your pallas kernel here")


# speedup vs baseline: 3.2255x; 3.2255x over previous
import functools

import jax
import jax.numpy as jnp
from jax import lax
from jax.experimental import pallas as pl
from jax.experimental.pallas import tpu as pltpu
from jax.experimental.pallas import tpu_sc as plsc

T, N, F_IN, F_ST, H, E = 12, 10000, 16, 8, 128, 320000
HD2 = H // 2

NC, NS = 2, 16
NW = NC * NS
EPW = E // NW
SC_B = 80
SC_NB = EPW // SC_B
RPT = N // NS
RPT8 = 1000


def _make_spmm(D):
    mesh = plsc.VectorSubcoreMesh(
        core_axis_name="c", subcore_axis_name="s", num_cores=NC, num_subcores=NS)

    @functools.partial(
        pl.kernel,
        out_type=jax.ShapeDtypeStruct((NC, N, D), jnp.float32),
        mesh=mesh,
        scratch_types=[
            pltpu.VMEM((EPW,), jnp.int32),
            pltpu.VMEM((EPW,), jnp.float32),
            pltpu.VMEM((SC_B,), jnp.int32),
            pltpu.VMEM((SC_B, D), jnp.float32),
            pltpu.VMEM_SHARED((N, D), jnp.float32),
            pltpu.SemaphoreType.DMA,
        ],
    )
    def spmm(x_hbm, rows_hbm, cols_hbm, vals_hbm, zeros_hbm, out_hbm,
             colv, valv, dstv, rowsv, acc, sem):
        c = lax.axis_index("c")
        s = lax.axis_index("s")
        w = c * NS + s
        e0 = pl.multiple_of(w * EPW, 8)
        pltpu.sync_copy(cols_hbm.at[pl.ds(e0, EPW)], colv)
        pltpu.sync_copy(vals_hbm.at[pl.ds(e0, EPW)], valv)
        r0 = pl.multiple_of(s * RPT, 8)

        @pl.when(s < N // RPT8)
        def _zero():
            z0 = pl.multiple_of(s * RPT8, 8)
            pltpu.sync_copy(zeros_hbm.at[pl.ds(0, RPT8)], acc.at[pl.ds(z0, RPT8)])

        plsc.subcore_barrier()

        def batch(i, carry):
            b0 = pl.multiple_of(e0 + i * SC_B, 8)
            pltpu.sync_copy(rows_hbm.at[pl.ds(b0, SC_B)], dstv)
            pltpu.async_copy(
                x_hbm.at[colv.at[pl.ds(i * SC_B, SC_B)]], rowsv, sem).wait()

            def scale16(g, carry2):
                val16 = valv[pl.ds(i * SC_B + g * 16, 16)]
                for k in range(16):
                    v = val16[k]
                    j = g * 16 + k
                    for q in range(D // 16):
                        rowsv[j, pl.ds(16 * q, 16)] = rowsv[j, pl.ds(16 * q, 16)] * v
                return carry2

            lax.fori_loop(0, SC_B // 16, scale16, 0)
            pltpu.sync_copy(rowsv, acc.at[dstv], add=True)
            return carry

        lax.fori_loop(0, SC_NB, batch, 0)
        plsc.subcore_barrier()

        @pl.when(s < N // RPT8)
        def _copy_out():
            z0 = pl.multiple_of(s * RPT8, 8)
            pltpu.sync_copy(acc.at[pl.ds(z0, RPT8)], out_hbm.at[c, pl.ds(z0, RPT8)])

    return spmm


_spmm128 = _make_spmm(H)

RB = 1000
NRB = N // RB


def _ln_blk(x, g, b):
    mu = jnp.mean(x, axis=-1, keepdims=True)
    var = jnp.mean((x - mu) ** 2, axis=-1, keepdims=True)
    return g * (x - mu) * lax.rsqrt(var + 1e-5) + b


def _fuse_body(ns_ref, st_ref, wf_ref, bf_ref, gf_ref, bfn_ref, o_ref):
    ns = ns_ref[0]
    st = st_ref[...]
    y = (jnp.dot(ns, wf_ref[:F_IN, :], preferred_element_type=jnp.float32)
         + jnp.dot(st, wf_ref[F_IN:, :], preferred_element_type=jnp.float32)
         + bf_ref[...])
    y = _ln_blk(y, gf_ref[...], bfn_ref[...])
    o_ref[0] = 0.5 * y * (1.0 + lax.erf(y / jnp.sqrt(2.0).astype(jnp.float32)))


def _fuse(node_seq, static_feat, Wf, bf, gf, bfn):
    return pl.pallas_call(
        _fuse_body,
        grid=(T, NRB),
        in_specs=[
            pl.BlockSpec((1, RB, F_IN), lambda t, i: (t, i, 0)),
            pl.BlockSpec((RB, F_ST), lambda t, i: (i, 0)),
            pl.BlockSpec((F_IN + F_ST, HD2), lambda t, i: (0, 0)),
            pl.BlockSpec((1, HD2), lambda t, i: (0, 0)),
            pl.BlockSpec((1, HD2), lambda t, i: (0, 0)),
            pl.BlockSpec((1, HD2), lambda t, i: (0, 0)),
        ],
        out_specs=pl.BlockSpec((1, RB, HD2), lambda t, i: (t, i, 0)),
        out_shape=jax.ShapeDtypeStruct((T, N, HD2), jnp.float32),
    )(node_seq, static_feat, Wf, bf.reshape(1, -1), gf.reshape(1, -1),
      bfn.reshape(1, -1))


def _gate_body(dx, x_ref, h_ref, wg_ref, bg_ref, lo_ref, hi_ref):
    g = (jnp.dot(x_ref[...], wg_ref[:dx, :], preferred_element_type=jnp.float32)
         + jnp.dot(h_ref[...], wg_ref[dx:, :], preferred_element_type=jnp.float32)
         + bg_ref[...])
    lo_ref[...] = g[:, :H]
    hi_ref[...] = g[:, H:]


def _gate(x, h, Wg, bg, dx):
    return pl.pallas_call(
        functools.partial(_gate_body, dx),
        grid=(NRB,),
        in_specs=[
            pl.BlockSpec((RB, dx), lambda i: (i, 0)),
            pl.BlockSpec((RB, H), lambda i: (i, 0)),
            pl.BlockSpec((dx + H, 2 * H), lambda i: (0, 0)),
            pl.BlockSpec((1, 2 * H), lambda i: (0, 0)),
        ],
        out_specs=[
            pl.BlockSpec((RB, H), lambda i: (i, 0)),
            pl.BlockSpec((RB, H), lambda i: (i, 0)),
        ],
        out_shape=[
            jax.ShapeDtypeStruct((N, H), jnp.float32),
            jax.ShapeDtypeStruct((N, H), jnp.float32),
        ],
    )(x, h, Wg, bg.reshape(1, -1))


def _mid_body(dx, glo_ref, ghi_ref, x_ref, h_ref, wc_ref, bc_ref, c_ref, u_ref):
    r = jax.nn.sigmoid(glo_ref[0] + glo_ref[1])
    u = jax.nn.sigmoid(ghi_ref[0] + ghi_ref[1])
    rh = r * h_ref[...]
    c_ref[...] = (jnp.dot(x_ref[...], wc_ref[:dx, :], preferred_element_type=jnp.float32)
                  + jnp.dot(rh, wc_ref[dx:, :], preferred_element_type=jnp.float32)
                  + bc_ref[...])
    u_ref[...] = u


def _mid(glo_p, ghi_p, x, h, Wc, bc, dx):
    return pl.pallas_call(
        functools.partial(_mid_body, dx),
        grid=(NRB,),
        in_specs=[
            pl.BlockSpec((NC, RB, H), lambda i: (0, i, 0)),
            pl.BlockSpec((NC, RB, H), lambda i: (0, i, 0)),
            pl.BlockSpec((RB, dx), lambda i: (i, 0)),
            pl.BlockSpec((RB, H), lambda i: (i, 0)),
            pl.BlockSpec((dx + H, H), lambda i: (0, 0)),
            pl.BlockSpec((1, H), lambda i: (0, 0)),
        ],
        out_specs=[
            pl.BlockSpec((RB, H), lambda i: (i, 0)),
            pl.BlockSpec((RB, H), lambda i: (i, 0)),
        ],
        out_shape=[
            jax.ShapeDtypeStruct((N, H), jnp.float32),
            jax.ShapeDtypeStruct((N, H), jnp.float32),
        ],
    )(glo_p, ghi_p, x, h, Wc, bc.reshape(1, -1))


def _fin_body(cp_ref, u_ref, h_ref, gn_ref, bn_ref, hn_ref):
    c = jnp.tanh(cp_ref[0] + cp_ref[1])
    u = u_ref[...]
    z = u * h_ref[...] + (1.0 - u) * c
    hn_ref[...] = _ln_blk(z, gn_ref[...], bn_ref[...])


def _fin(cp, u, h, gn, bn):
    return pl.pallas_call(
        _fin_body,
        grid=(NRB,),
        in_specs=[
            pl.BlockSpec((NC, RB, H), lambda i: (0, i, 0)),
            pl.BlockSpec((RB, H), lambda i: (i, 0)),
            pl.BlockSpec((RB, H), lambda i: (i, 0)),
            pl.BlockSpec((1, H), lambda i: (0, 0)),
            pl.BlockSpec((1, H), lambda i: (0, 0)),
        ],
        out_specs=pl.BlockSpec((RB, H), lambda i: (i, 0)),
        out_shape=jax.ShapeDtypeStruct((N, H), jnp.float32),
    )(cp, u, h, gn.reshape(1, -1), bn.reshape(1, -1))


def kernel(node_seq, static_feat, adj_vals, Wf, bf, gf, bfn, Wg0, bg0, Wc0, bc0,
           gn0, bn0, Wg1, bg1, Wc1, bc1, gn1, bn1, edge_index):
    rows = edge_index[0]
    cols = edge_index[1]
    zeros = jnp.zeros((RPT8, H), jnp.float32)

    def spmm(X):
        return _spmm128(X, rows, cols, adj_vals, zeros)

    xall = _fuse(node_seq, static_feat, Wf, bf, gf, bfn)
    layers = [(Wg0, bg0, Wc0, bc0, gn0, bn0), (Wg1, bg1, Wc1, bc1, gn1, bn1)]
    h = [jnp.zeros((N, H), jnp.float32) for _ in layers]
    for t in range(T):
        x = xall[t]
        for i, (Wg, bg, Wc, bc, gn, bn) in enumerate(layers):
            dx = x.shape[1]
            glo, ghi = _gate(x, h[i], Wg, bg, dx)
            glo_p = spmm(glo)
            ghi_p = spmm(ghi)
            cpre, u = _mid(glo_p, ghi_p, x, h[i], Wc, bc, dx)
            cp = spmm(cpre)
            hn = _fin(cp, u, h[i], gn, bn)
            h[i] = hn
            x = hn
    return jnp.stack(h)


# speedup vs baseline: 5.5921x; 1.7337x over previous
import functools

import jax
import jax.numpy as jnp
from jax import lax
from jax.experimental import pallas as pl
from jax.experimental.pallas import tpu as pltpu
from jax.experimental.pallas import tpu_sc as plsc

T, N, F_IN, F_ST, H, E = 12, 10000, 16, 8, 128, 320000
HD2 = H // 2

NC, NS = 2, 16
NW = NC * NS
EPW = E // NW
SC_B = 80
SC_NB = EPW // SC_B
SCH = 25
NSCH = SC_NB // SCH
RPT8 = 1000


def _make_spmm(npass):
    mesh = plsc.VectorSubcoreMesh(
        core_axis_name="c", subcore_axis_name="s", num_cores=NC, num_subcores=NS)

    @functools.partial(
        pl.kernel,
        out_type=jax.ShapeDtypeStruct((NC, N, npass * H), jnp.float32),
        mesh=mesh,
        scratch_types=[
            pltpu.VMEM((SCH * SC_B,), jnp.int32),
            pltpu.VMEM((SCH * SC_B,), jnp.float32),
            pltpu.VMEM((SCH, SC_B), jnp.int32),
            pltpu.VMEM((2, SC_B, H), jnp.float32),
            pltpu.VMEM_SHARED((N, H), jnp.float32),
            pltpu.SemaphoreType.DMA,
            pltpu.SemaphoreType.DMA,
        ],
    )
    def spmm(*refs):
        xs = refs[:npass]
        rows_hbm, cols_hbm, vals_hbm, zeros_hbm, out_hbm = refs[npass:npass + 5]
        colv, valv, dstm, rows2, acc = refs[npass + 5:npass + 10]
        gsem = refs[npass + 10:npass + 12]
        c = lax.axis_index("c")
        s = lax.axis_index("s")
        w = c * NS + s
        z0 = pl.multiple_of(s * RPT8, 8)

        for p in range(npass):
            x_ref = xs[p]

            @pl.when(s < N // RPT8)
            def _zero():
                pltpu.sync_copy(zeros_hbm.at[pl.ds(0, RPT8)], acc.at[pl.ds(z0, RPT8)])

            plsc.subcore_barrier()

            def gather(i, hb):
                pltpu.async_copy(
                    x_ref.at[colv.at[pl.ds(i * SC_B, SC_B)]], rows2.at[hb],
                    gsem[hb])

            def gwait(hb):
                pltpu.make_async_copy(
                    x_ref.at[colv.at[pl.ds(0, SC_B)]], rows2.at[hb],
                    gsem[hb]).wait()

            def process(i, hb):
                gwait(hb)

                @pl.when(i + 1 < SCH)
                def _next():
                    gather(i + 1, 1 - hb)

                def scale16(g, carry2):
                    val16 = valv[pl.ds(i * SC_B + g * 16, 16)]
                    for k in range(16):
                        v = val16[k]
                        for q in range(H // 16):
                            rows2[hb, g * 16 + k, pl.ds(16 * q, 16)] = (
                                rows2[hb, g * 16 + k, pl.ds(16 * q, 16)] * v)
                    return carry2

                lax.fori_loop(0, SC_B // 16, scale16, 0)
                pltpu.sync_copy(rows2.at[hb], acc.at[dstm.at[i]], add=True)

            def superchunk(sc, carry):
                e0 = pl.multiple_of(w * EPW + sc * (SCH * SC_B), 8)
                pltpu.sync_copy(cols_hbm.at[pl.ds(e0, SCH * SC_B)], colv)
                pltpu.sync_copy(vals_hbm.at[pl.ds(e0, SCH * SC_B)], valv)
                pltpu.sync_copy(rows_hbm.at[w, sc], dstm)
                gather(0, 0)

                def pair(k2, carry2):
                    process(2 * k2, 0)
                    process(2 * k2 + 1, 1)
                    return carry2

                lax.fori_loop(0, SCH // 2, pair, 0)
                process(SCH - 1, 0)
                return carry

            lax.fori_loop(0, NSCH, superchunk, 0)
            plsc.subcore_barrier()

            @pl.when(s < N // RPT8)
            def _copy_out():
                pltpu.sync_copy(acc.at[pl.ds(z0, RPT8)],
                                out_hbm.at[c, pl.ds(z0, RPT8), pl.ds(p * H, H)])

            plsc.subcore_barrier()

    return spmm


_spmm1 = _make_spmm(1)
_spmm2 = _make_spmm(2)

RB = 1000
NRB = N // RB


def _ln_blk(x, g, b):
    mu = jnp.mean(x, axis=-1, keepdims=True)
    var = jnp.mean((x - mu) ** 2, axis=-1, keepdims=True)
    return g * (x - mu) * lax.rsqrt(var + 1e-5) + b


def _fuse_body(ns_ref, st_ref, wf_ref, bf_ref, gf_ref, bfn_ref, o_ref):
    ns = ns_ref[0]
    st = st_ref[...]
    y = (jnp.dot(ns, wf_ref[:F_IN, :], preferred_element_type=jnp.float32)
         + jnp.dot(st, wf_ref[F_IN:, :], preferred_element_type=jnp.float32)
         + bf_ref[...])
    y = _ln_blk(y, gf_ref[...], bfn_ref[...])
    o_ref[0] = 0.5 * y * (1.0 + lax.erf(y / jnp.sqrt(2.0).astype(jnp.float32)))


def _fuse(node_seq, static_feat, Wf, bf, gf, bfn):
    return pl.pallas_call(
        _fuse_body,
        grid=(T, NRB),
        in_specs=[
            pl.BlockSpec((1, RB, F_IN), lambda t, i: (t, i, 0)),
            pl.BlockSpec((RB, F_ST), lambda t, i: (i, 0)),
            pl.BlockSpec((F_IN + F_ST, HD2), lambda t, i: (0, 0)),
            pl.BlockSpec((1, HD2), lambda t, i: (0, 0)),
            pl.BlockSpec((1, HD2), lambda t, i: (0, 0)),
            pl.BlockSpec((1, HD2), lambda t, i: (0, 0)),
        ],
        out_specs=pl.BlockSpec((1, RB, HD2), lambda t, i: (t, i, 0)),
        out_shape=jax.ShapeDtypeStruct((T, N, HD2), jnp.float32),
    )(node_seq, static_feat, Wf, bf.reshape(1, -1), gf.reshape(1, -1),
      bfn.reshape(1, -1))


def _gate_body(dx, x_ref, h_ref, wg_ref, bg_ref, lo_ref, hi_ref):
    g = (jnp.dot(x_ref[...], wg_ref[:dx, :], preferred_element_type=jnp.float32)
         + jnp.dot(h_ref[...], wg_ref[dx:, :], preferred_element_type=jnp.float32)
         + bg_ref[...])
    lo_ref[...] = g[:, :H]
    hi_ref[...] = g[:, H:]


def _gate(x, h, Wg, bg, dx):
    return pl.pallas_call(
        functools.partial(_gate_body, dx),
        grid=(NRB,),
        in_specs=[
            pl.BlockSpec((RB, dx), lambda i: (i, 0)),
            pl.BlockSpec((RB, H), lambda i: (i, 0)),
            pl.BlockSpec((dx + H, 2 * H), lambda i: (0, 0)),
            pl.BlockSpec((1, 2 * H), lambda i: (0, 0)),
        ],
        out_specs=[
            pl.BlockSpec((RB, H), lambda i: (i, 0)),
            pl.BlockSpec((RB, H), lambda i: (i, 0)),
        ],
        out_shape=[
            jax.ShapeDtypeStruct((N, H), jnp.float32),
            jax.ShapeDtypeStruct((N, H), jnp.float32),
        ],
    )(x, h, Wg, bg.reshape(1, -1))


def _mid_body(dx, gp_ref, x_ref, h_ref, wc_ref, bc_ref, c_ref, u_ref):
    r = jax.nn.sigmoid(gp_ref[0, :, :H] + gp_ref[1, :, :H])
    u = jax.nn.sigmoid(gp_ref[0, :, H:] + gp_ref[1, :, H:])
    rh = r * h_ref[...]
    c_ref[...] = (jnp.dot(x_ref[...], wc_ref[:dx, :], preferred_element_type=jnp.float32)
                  + jnp.dot(rh, wc_ref[dx:, :], preferred_element_type=jnp.float32)
                  + bc_ref[...])
    u_ref[...] = u


def _mid(gp, x, h, Wc, bc, dx):
    return pl.pallas_call(
        functools.partial(_mid_body, dx),
        grid=(NRB,),
        in_specs=[
            pl.BlockSpec((NC, RB, 2 * H), lambda i: (0, i, 0)),
            pl.BlockSpec((RB, dx), lambda i: (i, 0)),
            pl.BlockSpec((RB, H), lambda i: (i, 0)),
            pl.BlockSpec((dx + H, H), lambda i: (0, 0)),
            pl.BlockSpec((1, H), lambda i: (0, 0)),
        ],
        out_specs=[
            pl.BlockSpec((RB, H), lambda i: (i, 0)),
            pl.BlockSpec((RB, H), lambda i: (i, 0)),
        ],
        out_shape=[
            jax.ShapeDtypeStruct((N, H), jnp.float32),
            jax.ShapeDtypeStruct((N, H), jnp.float32),
        ],
    )(gp, x, h, Wc, bc.reshape(1, -1))


def _hn_from(cp_ref, u_ref, h_ref, gn_ref, bn_ref):
    c = jnp.tanh(cp_ref[0] + cp_ref[1])
    u = u_ref[...]
    z = u * h_ref[...] + (1.0 - u) * c
    return _ln_blk(z, gn_ref[...], bn_ref[...])


def _fin_body(cp_ref, u_ref, h_ref, gn_ref, bn_ref, hn_ref):
    hn_ref[...] = _hn_from(cp_ref, u_ref, h_ref, gn_ref, bn_ref)


def _fin(cp, u, h, gn, bn):
    return pl.pallas_call(
        _fin_body,
        grid=(NRB,),
        in_specs=[
            pl.BlockSpec((NC, RB, H), lambda i: (0, i, 0)),
            pl.BlockSpec((RB, H), lambda i: (i, 0)),
            pl.BlockSpec((RB, H), lambda i: (i, 0)),
            pl.BlockSpec((1, H), lambda i: (0, 0)),
            pl.BlockSpec((1, H), lambda i: (0, 0)),
        ],
        out_specs=pl.BlockSpec((RB, H), lambda i: (i, 0)),
        out_shape=jax.ShapeDtypeStruct((N, H), jnp.float32),
    )(cp, u, h, gn.reshape(1, -1), bn.reshape(1, -1))


def _fingate_body(dx, use_hn_as_x, cp_ref, u_ref, h_ref, gn_ref, bn_ref,
                  xn_ref, hx_ref, wg_ref, bg_ref, hn_ref, lo_ref, hi_ref):
    hn = _hn_from(cp_ref, u_ref, h_ref, gn_ref, bn_ref)
    hn_ref[...] = hn
    x = hn if use_hn_as_x else xn_ref[...]
    g = (jnp.dot(x, wg_ref[:dx, :], preferred_element_type=jnp.float32)
         + jnp.dot(hx_ref[...], wg_ref[dx:, :], preferred_element_type=jnp.float32)
         + bg_ref[...])
    lo_ref[...] = g[:, :H]
    hi_ref[...] = g[:, H:]


def _fingate(cp, u, h, gn, bn, xn, hx, Wg, bg, dx, use_hn_as_x):
    return pl.pallas_call(
        functools.partial(_fingate_body, dx, use_hn_as_x),
        grid=(NRB,),
        in_specs=[
            pl.BlockSpec((NC, RB, H), lambda i: (0, i, 0)),
            pl.BlockSpec((RB, H), lambda i: (i, 0)),
            pl.BlockSpec((RB, H), lambda i: (i, 0)),
            pl.BlockSpec((1, H), lambda i: (0, 0)),
            pl.BlockSpec((1, H), lambda i: (0, 0)),
            pl.BlockSpec((RB, dx), lambda i: (i, 0)),
            pl.BlockSpec((RB, H), lambda i: (i, 0)),
            pl.BlockSpec((dx + H, 2 * H), lambda i: (0, 0)),
            pl.BlockSpec((1, 2 * H), lambda i: (0, 0)),
        ],
        out_specs=[
            pl.BlockSpec((RB, H), lambda i: (i, 0)),
            pl.BlockSpec((RB, H), lambda i: (i, 0)),
            pl.BlockSpec((RB, H), lambda i: (i, 0)),
        ],
        out_shape=[
            jax.ShapeDtypeStruct((N, H), jnp.float32),
            jax.ShapeDtypeStruct((N, H), jnp.float32),
            jax.ShapeDtypeStruct((N, H), jnp.float32),
        ],
    )(cp, u, h, gn.reshape(1, -1), bn.reshape(1, -1), xn, hx, Wg,
      bg.reshape(1, -1))


def kernel(node_seq, static_feat, adj_vals, Wf, bf, gf, bfn, Wg0, bg0, Wc0, bc0,
           gn0, bn0, Wg1, bg1, Wc1, bc1, gn1, bn1, edge_index):
    rows3d = edge_index[0].reshape(NW, NSCH, SCH, SC_B)
    cols = edge_index[1]
    zeros = jnp.zeros((RPT8, H), jnp.float32)

    xall = _fuse(node_seq, static_feat, Wf, bf, gf, bfn)
    h0 = jnp.zeros((N, H), jnp.float32)
    h1 = jnp.zeros((N, H), jnp.float32)
    glo, ghi = _gate(xall[0], h0, Wg0, bg0, HD2)
    for t in range(T):
        gp = _spmm2(glo, ghi, rows3d, cols, adj_vals, zeros)
        cpre, u = _mid(gp, xall[t], h0, Wc0, bc0, HD2)
        cp = _spmm1(cpre, rows3d, cols, adj_vals, zeros)
        h0, glo, ghi = _fingate(cp, u, h0, gn0, bn0, h0, h1, Wg1, bg1, H, True)
        gp = _spmm2(glo, ghi, rows3d, cols, adj_vals, zeros)
        cpre, u = _mid(gp, h0, h1, Wc1, bc1, H)
        cp = _spmm1(cpre, rows3d, cols, adj_vals, zeros)
        if t < T - 1:
            h1, glo, ghi = _fingate(cp, u, h1, gn1, bn1, xall[t + 1], h0,
                                    Wg0, bg0, HD2, False)
        else:
            h1 = _fin(cp, u, h1, gn1, bn1)
    return jnp.stack([h0, h1])
